# Initial kernel scaffold; baseline (speedup 1.0000x reference)
import jax
import jax.numpy as jnp
from jax.experimental import pallas as pl


def kernel(flow, masks):
    raise NotImplementedError("write your pallas kernel here")


# speedup vs baseline: 57.0794x; 57.0794x over previous
import functools

import jax
import jax.numpy as jnp
from jax import lax
from jax.experimental import pallas as pl
from jax.experimental.pallas import tpu as pltpu
from jax.experimental.pallas import tpu_sc as plsc

NUM_SEGMENTS = 16
MIN_PIXELS = 50

NC = 2
NS = 16
L = 16
NW = NC * NS

STATS = 4
LANE_TBL = NUM_SEGMENTS * STATS
TBL = L * LANE_TBL

N_PIX = 8 * 512 * 512
N_TILE = N_PIX // NW
BLK = 8192
N_BLK = N_TILE // BLK
VECS = BLK // L
UNROLL = 4


def _make_sc_kernel():
    mesh = plsc.VectorSubcoreMesh(
        core_axis_name="c", subcore_axis_name="s", num_cores=NC, num_subcores=NS
    )

    @functools.partial(
        pl.kernel,
        out_type=jax.ShapeDtypeStruct((NW, LANE_TBL), jnp.float32),
        mesh=mesh,
        compiler_params=pltpu.CompilerParams(needs_layout_passes=False),
        scratch_types=[
            pltpu.VMEM((BLK,), jnp.float32),
            pltpu.VMEM((BLK,), jnp.float32),
            pltpu.VMEM((BLK,), jnp.int32),
            pltpu.VMEM((TBL,), jnp.float32),
            pltpu.VMEM((LANE_TBL,), jnp.float32),
        ],
    )
    def seg_stats(u_hbm, v_hbm, m_hbm, out_hbm, u_v, v_v, m_v, acc, fold):
        wid = lax.axis_index("s") * NC + lax.axis_index("c")
        base = wid * N_TILE

        zeros = jnp.zeros((L,), jnp.float32)
        for i in range(TBL // L):
            acc[pl.ds(i * L, L)] = zeros

        lane_base = lax.iota(jnp.int32, L) * LANE_TBL
        ones = jnp.ones((L,), jnp.float32)

        def body(i, carry):
            for k in range(UNROLL):
                off = (i * UNROLL + k) * L
                m = m_v[pl.ds(off, L)]
                u = u_v[pl.ds(off, L)]
                v = v_v[pl.ds(off, L)]
                idx0 = lane_base + m * STATS
                sq = u * u + v * v
                plsc.addupdate_scatter(acc, [idx0], ones)
                plsc.addupdate_scatter(acc, [idx0 + 1], u)
                plsc.addupdate_scatter(acc, [idx0 + 2], v)
                plsc.addupdate_scatter(acc, [idx0 + 3], sq)
            return carry

        for blk in range(N_BLK):
            start = base + blk * BLK
            pltpu.sync_copy(u_hbm.at[pl.ds(start, BLK)], u_v)
            pltpu.sync_copy(v_hbm.at[pl.ds(start, BLK)], v_v)
            pltpu.sync_copy(m_hbm.at[pl.ds(start, BLK)], m_v)
            lax.fori_loop(0, VECS // UNROLL, body, 0, unroll=False)

        for c in range(LANE_TBL // L):
            tot = acc[pl.ds(c * L, L)]
            for lane in range(1, L):
                tot = tot + acc[pl.ds(lane * LANE_TBL + c * L, L)]
            fold[pl.ds(c * L, L)] = tot

        pltpu.sync_copy(fold, out_hbm.at[wid])

    return seg_stats


_seg_stats = _make_sc_kernel()


def kernel(flow, masks):
    u = flow[:, 0].reshape(-1)
    v = flow[:, 1].reshape(-1)
    m = masks.reshape(-1).astype(jnp.int32)

    part = _seg_stats(u, v, m)

    stats = part.reshape(8, NW // 8, NUM_SEGMENTS, STATS).sum(axis=1)
    stats = stats.reshape(8 * NUM_SEGMENTS, STATS)
    cnt = stats[:, 0]
    su = stats[:, 1]
    sv = stats[:, 2]
    ssq = stats[:, 3]
    safe_cnt = jnp.maximum(cnt, 1.0)
    denom = jnp.maximum(cnt - 1.0, 1.0)
    var_sum = (ssq - (su * su + sv * sv) / safe_cnt) / denom
    seg_local = jnp.arange(8 * NUM_SEGMENTS, dtype=jnp.int32) % NUM_SEGMENTS
    valid = (seg_local != 0) & (cnt >= MIN_PIXELS)
    num_valid = jnp.sum(valid)
    total_loss = jnp.sum(jnp.where(valid, var_sum, 0.0))
    return jnp.where(
        num_valid > 0,
        total_loss / jnp.maximum(num_valid, 1).astype(flow.dtype),
        jnp.array(0.0, dtype=flow.dtype),
    )
